# baseline (speedup 1.0000x reference)
import functools

import jax
import jax.numpy as jnp
from jax import lax
from jax.experimental import pallas as pl
from jax.experimental.pallas import tpu as pltpu
from jax.experimental.pallas import tpu_sc as plsc

_N = 10000
_E = 320000
_NP = 4
_EH = _E // _NP
_D = 128
_H = 64
_CH = 128
_NSUB = 16
_NCHUNKS = _EH // _CH
_CPT = (_NCHUNKS + _NSUB - 1) // _NSUB
_NB = 2
_CPT2 = ((_CPT + _NB - 1) // _NB) * _NB
_ROWS_A = 640
_ROWS_LAST = _N - _ROWS_A * (_NSUB - 1)

_HP = jax.lax.Precision.HIGHEST


def _hid_body(x_ref, wlt_ref, b_ref, h0_ref, h1_ref):
    acc = jnp.dot(x_ref[...], wlt_ref[...],
                  preferred_element_type=jnp.float32, precision=_HP)
    acc = acc + b_ref[...]
    h0_ref[...] = acc[:, :_H]
    h1_ref[...] = acc[:, _H:]


_hid_call = pl.pallas_call(
    _hid_body,
    grid=(5,),
    in_specs=[
        pl.BlockSpec((2000, _D), lambda i: (i, 0)),
        pl.BlockSpec((_D, _D), lambda i: (0, 0)),
        pl.BlockSpec((1, _D), lambda i: (0, 0)),
    ],
    out_specs=[
        pl.BlockSpec((2000, _H), lambda i: (i, 0)),
        pl.BlockSpec((2000, _H), lambda i: (i, 0)),
    ],
    out_shape=[jax.ShapeDtypeStruct((_N, _H), jnp.float32)] * 2,
)


_EB = 8000


def _edge_body(c_ref, w1_ref, b1_ref, w2t_ref, b2_ref, e_ref):
    c = c_ref[...]
    h = c * w1_ref[...] + b1_ref[...]
    h = jnp.where(h >= 0, h, 0.2 * h)
    nc = jnp.dot(h, w2t_ref[...],
                 preferred_element_type=jnp.float32, precision=_HP)
    nc = nc + b2_ref[...]
    e_ref[...] = jnp.exp(nc)


_edge_call = pl.pallas_call(
    _edge_body,
    grid=(_EH // _EB,),
    in_specs=[
        pl.BlockSpec((_EB, 1), lambda i: (i, 0)),
        pl.BlockSpec((1, _D), lambda i: (0, 0)),
        pl.BlockSpec((1, _D), lambda i: (0, 0)),
        pl.BlockSpec((_D, _D), lambda i: (0, 0)),
        pl.BlockSpec((1, _D), lambda i: (0, 0)),
    ],
    out_specs=pl.BlockSpec((_EB, _D), lambda i: (i, 0)),
    out_shape=jax.ShapeDtypeStruct((_EH, _D), jnp.float32),
)


@functools.cache
def _make_sc_call(eoff):
    mesh = plsc.VectorSubcoreMesh(core_axis_name="c", subcore_axis_name="s")

    @functools.partial(
        pl.kernel,
        out_type=[jax.ShapeDtypeStruct((_N, _D), jnp.float32),
                  jax.ShapeDtypeStruct((_N, _D), jnp.float32)],
        mesh=mesh,
        compiler_params=pltpu.CompilerParams(use_tc_tiling_on_sc=False),
        scratch_types=[
            pltpu.VMEM((_CH,), jnp.int32),
            pltpu.VMEM((_CH,), jnp.int32),
            pltpu.VMEM((_CH,), jnp.int32),
            pltpu.VMEM((_CH,), jnp.int32),
            pltpu.VMEM((_CH, _H), jnp.float32),
            pltpu.VMEM((_CH, _H), jnp.float32),
            pltpu.VMEM((_CH, _H), jnp.float32),
            pltpu.VMEM((_CH, _H), jnp.float32),
            pltpu.VMEM((_CH, _H), jnp.float32),
            pltpu.VMEM_SHARED((_N, _H), jnp.float32),
            pltpu.VMEM_SHARED((_N, _H), jnp.float32),
            pltpu.SemaphoreType.DMA,
            pltpu.SemaphoreType.DMA,
            pltpu.SemaphoreType.DMA,
            pltpu.SemaphoreType.DMA,
        ],
    )
    def sc_call(ei_hbm, ej_hbm, e_hbm, h0_hbm, h1_hbm, zero_hbm,
                u_hbm, s_hbm,
                idxg0, idxg1, idxs0, idxs1, hbuf0, hbuf1, ebuf0, ebuf1,
                pbuf, uacc, sacc, sem_e0, sem_e1, sem_h0, sem_h1):
        idxg = (idxg0, idxg1)
        idxs = (idxs0, idxs1)
        hbuf = (hbuf0, hbuf1)
        ebuf = (ebuf0, ebuf1)
        sem_e = (sem_e0, sem_e1)
        sem_h = (sem_h0, sem_h1)
        cid = lax.axis_index("c")
        sid = lax.axis_index("s")
        row0 = sid * _ROWS_A
        col0 = cid * _H

        @pl.when(sid < _NSUB - 1)
        def _():
            pltpu.sync_copy(zero_hbm.at[pl.ds(row0, _ROWS_A)],
                            uacc.at[pl.ds(row0, _ROWS_A)])
            pltpu.sync_copy(zero_hbm.at[pl.ds(row0, _ROWS_A)],
                            sacc.at[pl.ds(row0, _ROWS_A)])

        @pl.when(sid == _NSUB - 1)
        def _():
            pltpu.sync_copy(zero_hbm.at[pl.ds(row0, _ROWS_LAST)],
                            uacc.at[pl.ds(row0, _ROWS_LAST)])
            pltpu.sync_copy(zero_hbm.at[pl.ds(row0, _ROWS_LAST)],
                            sacc.at[pl.ds(row0, _ROWS_LAST)])

        plsc.subcore_barrier()

        def run(h_hbm):
            def issue(it, b):
                k = it * _NSUB + sid

                @pl.when(k < _NCHUNKS)
                def _():
                    base = k * _CH
                    pltpu.sync_copy(ej_hbm.at[pl.ds(eoff + base, _CH)],
                                    idxg[b])
                    pltpu.sync_copy(ei_hbm.at[pl.ds(eoff + base, _CH)],
                                    idxs[b])
                    pltpu.async_copy(
                        e_hbm.at[pl.ds(base, _CH), pl.ds(col0, _H)], ebuf[b],
                        sem_e[b])
                    pltpu.async_copy(h_hbm.at[idxg[b]], hbuf[b], sem_h[b])

            def process(it, b):
                k = it * _NSUB + sid

                @pl.when(k < _NCHUNKS)
                def _():
                    pltpu.make_async_copy(
                        e_hbm.at[pl.ds(0, _CH), pl.ds(col0, _H)], ebuf[b],
                        sem_e[b]).wait()
                    pltpu.make_async_copy(
                        h_hbm.at[idxg[b]], hbuf[b], sem_h[b]).wait()

                    @pl.loop(0, _CH)
                    def _(r):
                        for q in range(_H // 16):
                            sl = pl.ds(q * 16, 16)
                            pbuf[r, sl] = hbuf[b][r, sl] * ebuf[b][r, sl]

                    pltpu.sync_copy(ebuf[b], sacc.at[idxs[b]], add=True)
                    pltpu.sync_copy(pbuf, uacc.at[idxs[b]], add=True)

            for b in range(_NB):
                issue(b, b)

            @pl.loop(0, _CPT2, step=_NB)
            def _(it):
                for b in range(_NB):
                    process(it + b, b)
                    issue(it + b + _NB, b)

        @pl.when(cid == 0)
        def _():
            run(h0_hbm)

        @pl.when(cid == 1)
        def _():
            run(h1_hbm)

        plsc.subcore_barrier()

        @pl.when(sid < _NSUB - 1)
        def _():
            pltpu.sync_copy(uacc.at[pl.ds(row0, _ROWS_A)],
                            u_hbm.at[pl.ds(row0, _ROWS_A), pl.ds(col0, _H)])
            pltpu.sync_copy(sacc.at[pl.ds(row0, _ROWS_A)],
                            s_hbm.at[pl.ds(row0, _ROWS_A), pl.ds(col0, _H)])

        @pl.when(sid == _NSUB - 1)
        def _():
            pltpu.sync_copy(uacc.at[pl.ds(row0, _ROWS_LAST)],
                            u_hbm.at[pl.ds(row0, _ROWS_LAST), pl.ds(col0, _H)])
            pltpu.sync_copy(sacc.at[pl.ds(row0, _ROWS_LAST)],
                            s_hbm.at[pl.ds(row0, _ROWS_LAST), pl.ds(col0, _H)])

    return sc_call


_FB = 2000


def _fin_body(*refs):
    o_ref = refs[-1]
    u = refs[0][...]
    s = refs[1][...]
    for p in range(1, _NP):
        u = u + refs[2 * p][...]
        s = s + refs[2 * p + 1][...]
    r = u / (s + 1e-16)
    o_ref[...] = jnp.maximum(r, 0.0)


_fin_call = pl.pallas_call(
    _fin_body,
    grid=(_N // _FB,),
    in_specs=[pl.BlockSpec((_FB, _D), lambda i: (i, 0))] * (2 * _NP),
    out_specs=pl.BlockSpec((_FB, _D), lambda i: (i, 0)),
    out_shape=jax.ShapeDtypeStruct((_N, _D), jnp.float32),
)


def kernel(x, edge_index, curvature, W_lin, b_lin, W1, b1, W2, b2):
    ei = edge_index[0]
    ej = edge_index[1]
    h0, h1 = _hid_call(x, W_lin.T, b_lin.reshape(1, _D))
    w1r = W1.reshape(1, _D)
    b1r = b1.reshape(1, _D)
    w2t = W2.T
    b2r = b2.reshape(1, _D)
    zeros = jnp.zeros((_N, _H), jnp.float32)
    es = []
    for p in range(_NP):
        cp = lax.slice(curvature, (p * _EH, 0), ((p + 1) * _EH, 1))
        es.append(_edge_call(cp, w1r, b1r, w2t, b2r))
    parts = []
    eip, ejp = ei, ej
    for p in range(_NP):
        ep = es[p]
        if p:
            pu, ps = parts[-1]
            pu, ps, eip, ejp, ep = lax.optimization_barrier(
                (pu, ps, ei, ej, ep))
            parts[-1] = (pu, ps)
        parts.append(_make_sc_call(p * _EH)(eip, ejp, ep, h0, h1, zeros))
    flat = [a for pair in parts for a in pair]
    return _fin_call(*flat)


# speedup vs baseline: 1.2746x; 1.2746x over previous
import functools

import jax
import jax.numpy as jnp
from jax import lax
from jax.experimental import pallas as pl
from jax.experimental.pallas import tpu as pltpu
from jax.experimental.pallas import tpu_sc as plsc

_N = 10000
_E = 320000
_NP = 4
_EH = _E // _NP
_D = 128
_H = 64
_CH = 128
_NSUB = 16
_NCHUNKS = _EH // _CH
_CPT = (_NCHUNKS + _NSUB - 1) // _NSUB
_NB = 2
_CPT2 = ((_CPT + _NB - 1) // _NB) * _NB
_ROWS_A = 640
_ROWS_LAST = _N - _ROWS_A * (_NSUB - 1)

_HP = jax.lax.Precision.HIGHEST


def _hid_body(x_ref, wlt_ref, b_ref, h0_ref, h1_ref):
    acc = jnp.dot(x_ref[...], wlt_ref[...],
                  preferred_element_type=jnp.float32, precision=_HP)
    acc = acc + b_ref[...]
    h0_ref[...] = acc[:, :_H]
    h1_ref[...] = acc[:, _H:]


_hid_call = pl.pallas_call(
    _hid_body,
    grid=(5,),
    in_specs=[
        pl.BlockSpec((2000, _D), lambda i: (i, 0)),
        pl.BlockSpec((_D, _D), lambda i: (0, 0)),
        pl.BlockSpec((1, _D), lambda i: (0, 0)),
    ],
    out_specs=[
        pl.BlockSpec((2000, _H), lambda i: (i, 0)),
        pl.BlockSpec((2000, _H), lambda i: (i, 0)),
    ],
    out_shape=[jax.ShapeDtypeStruct((_N, _H), jnp.float32)] * 2,
)


_EB = 8000


def _edge_body(c_ref, w1_ref, b1_ref, w2t_ref, b2_ref, e_ref):
    c = c_ref[...]
    h = c * w1_ref[...] + b1_ref[...]
    h = jnp.where(h >= 0, h, 0.2 * h)
    nc = jnp.dot(h, w2t_ref[...],
                 preferred_element_type=jnp.float32, precision=_HP)
    nc = nc + b2_ref[...]
    e_ref[...] = jnp.exp(nc)


_edge_call = pl.pallas_call(
    _edge_body,
    grid=(_EH // _EB,),
    in_specs=[
        pl.BlockSpec((_EB, 1), lambda i: (i, 0)),
        pl.BlockSpec((1, _D), lambda i: (0, 0)),
        pl.BlockSpec((1, _D), lambda i: (0, 0)),
        pl.BlockSpec((_D, _D), lambda i: (0, 0)),
        pl.BlockSpec((1, _D), lambda i: (0, 0)),
    ],
    out_specs=pl.BlockSpec((_EB, _D), lambda i: (i, 0)),
    out_shape=jax.ShapeDtypeStruct((_EH, _D), jnp.float32),
)


_CPT_LAST = _NCHUNKS - (_NSUB - 1) * _CPT


@functools.cache
def _make_sc_call(stage):
    crow0 = stage * _NCHUNKS
    mesh = plsc.VectorSubcoreMesh(core_axis_name="c", subcore_axis_name="s")

    @functools.partial(
        pl.kernel,
        out_type=[jax.ShapeDtypeStruct((_N, _D), jnp.float32),
                  jax.ShapeDtypeStruct((_N, _D), jnp.float32)],
        mesh=mesh,
        compiler_params=pltpu.CompilerParams(use_tc_tiling_on_sc=False),
        scratch_types=[
            pltpu.VMEM((_CPT, _CH), jnp.int32),
            pltpu.VMEM((_CPT, _CH), jnp.int32),
            pltpu.VMEM((_CH, _H), jnp.float32),
            pltpu.VMEM((_CH, _H), jnp.float32),
            pltpu.VMEM((_CH, _H), jnp.float32),
            pltpu.VMEM((_CH, _H), jnp.float32),
            pltpu.VMEM_SHARED((_N, _H), jnp.float32),
            pltpu.VMEM_SHARED((_N, _H), jnp.float32),
            pltpu.SemaphoreType.DMA,
            pltpu.SemaphoreType.DMA,
            pltpu.SemaphoreType.DMA,
            pltpu.SemaphoreType.DMA,
        ],
    )
    def sc_call(ei_hbm, ej_hbm, e_hbm, h0_hbm, h1_hbm, zero_hbm,
                u_hbm, s_hbm,
                gslab, sslab, hbuf0, hbuf1, ebuf0, ebuf1,
                uacc, sacc, sem_e0, sem_e1, sem_h0, sem_h1):
        hbuf = (hbuf0, hbuf1)
        ebuf = (ebuf0, ebuf1)
        sem_e = (sem_e0, sem_e1)
        sem_h = (sem_h0, sem_h1)
        cid = lax.axis_index("c")
        sid = lax.axis_index("s")
        row0 = sid * _ROWS_A
        col0 = cid * _H
        myrow = crow0 + sid * _CPT

        @pl.when(sid < _NSUB - 1)
        def _():
            pltpu.sync_copy(zero_hbm.at[pl.ds(row0, _ROWS_A)],
                            uacc.at[pl.ds(row0, _ROWS_A)])
            pltpu.sync_copy(zero_hbm.at[pl.ds(row0, _ROWS_A)],
                            sacc.at[pl.ds(row0, _ROWS_A)])
            pltpu.sync_copy(ej_hbm.at[pl.ds(myrow, _CPT)], gslab)
            pltpu.sync_copy(ei_hbm.at[pl.ds(myrow, _CPT)], sslab)

        @pl.when(sid == _NSUB - 1)
        def _():
            pltpu.sync_copy(zero_hbm.at[pl.ds(row0, _ROWS_LAST)],
                            uacc.at[pl.ds(row0, _ROWS_LAST)])
            pltpu.sync_copy(zero_hbm.at[pl.ds(row0, _ROWS_LAST)],
                            sacc.at[pl.ds(row0, _ROWS_LAST)])
            pltpu.sync_copy(ej_hbm.at[pl.ds(myrow, _CPT_LAST)],
                            gslab.at[pl.ds(0, _CPT_LAST)])
            pltpu.sync_copy(ei_hbm.at[pl.ds(myrow, _CPT_LAST)],
                            sslab.at[pl.ds(0, _CPT_LAST)])

        plsc.subcore_barrier()

        def run(h_hbm):
            def guard(it):
                k = sid * _CPT + it
                return (k < _NCHUNKS) & (it < _CPT)

            def issue(it, b):
                @pl.when(guard(it))
                def _():
                    ebase = (sid * _CPT + it) * _CH
                    pltpu.async_copy(
                        e_hbm.at[pl.ds(ebase, _CH), pl.ds(col0, _H)], ebuf[b],
                        sem_e[b])
                    pltpu.async_copy(h_hbm.at[gslab.at[it]], hbuf[b],
                                     sem_h[b])

            def process(it, b):
                @pl.when(guard(it))
                def _():
                    pltpu.make_async_copy(
                        e_hbm.at[pl.ds(0, _CH), pl.ds(col0, _H)], ebuf[b],
                        sem_e[b]).wait()
                    pltpu.make_async_copy(
                        h_hbm.at[gslab.at[it]], hbuf[b], sem_h[b]).wait()

                    @pl.loop(0, _CH)
                    def _(r):
                        for q in range(_H // 16):
                            sl = pl.ds(q * 16, 16)
                            hbuf[b][r, sl] = hbuf[b][r, sl] * ebuf[b][r, sl]

                    pltpu.sync_copy(ebuf[b], sacc.at[sslab.at[it]], add=True)
                    pltpu.sync_copy(hbuf[b], uacc.at[sslab.at[it]], add=True)

            for b in range(_NB):
                issue(b, b)

            @pl.loop(0, _CPT2, step=_NB)
            def _(it):
                for b in range(_NB):
                    process(it + b, b)
                    issue(it + b + _NB, b)

        @pl.when(cid == 0)
        def _():
            run(h0_hbm)

        @pl.when(cid == 1)
        def _():
            run(h1_hbm)

        plsc.subcore_barrier()

        @pl.when(sid < _NSUB - 1)
        def _():
            pltpu.sync_copy(uacc.at[pl.ds(row0, _ROWS_A)],
                            u_hbm.at[pl.ds(row0, _ROWS_A), pl.ds(col0, _H)])
            pltpu.sync_copy(sacc.at[pl.ds(row0, _ROWS_A)],
                            s_hbm.at[pl.ds(row0, _ROWS_A), pl.ds(col0, _H)])

        @pl.when(sid == _NSUB - 1)
        def _():
            pltpu.sync_copy(uacc.at[pl.ds(row0, _ROWS_LAST)],
                            u_hbm.at[pl.ds(row0, _ROWS_LAST), pl.ds(col0, _H)])
            pltpu.sync_copy(sacc.at[pl.ds(row0, _ROWS_LAST)],
                            s_hbm.at[pl.ds(row0, _ROWS_LAST), pl.ds(col0, _H)])

    return sc_call


_FB = 2000


def _fin_body(*refs):
    o_ref = refs[-1]
    u = refs[0][...]
    s = refs[1][...]
    for p in range(1, _NP):
        u = u + refs[2 * p][...]
        s = s + refs[2 * p + 1][...]
    r = u / (s + 1e-16)
    o_ref[...] = jnp.maximum(r, 0.0)


_fin_call = pl.pallas_call(
    _fin_body,
    grid=(_N // _FB,),
    in_specs=[pl.BlockSpec((_FB, _D), lambda i: (i, 0))] * (2 * _NP),
    out_specs=pl.BlockSpec((_FB, _D), lambda i: (i, 0)),
    out_shape=jax.ShapeDtypeStruct((_N, _D), jnp.float32),
)


def kernel(x, edge_index, curvature, W_lin, b_lin, W1, b1, W2, b2):
    ei = edge_index[0].reshape(_E // _CH, _CH)
    ej = edge_index[1].reshape(_E // _CH, _CH)
    h0, h1 = _hid_call(x, W_lin.T, b_lin.reshape(1, _D))
    w1r = W1.reshape(1, _D)
    b1r = b1.reshape(1, _D)
    w2t = W2.T
    b2r = b2.reshape(1, _D)
    zeros = jnp.zeros((_N, _H), jnp.float32)
    es = []
    for p in range(_NP):
        cp = lax.slice(curvature, (p * _EH, 0), ((p + 1) * _EH, 1))
        es.append(_edge_call(cp, w1r, b1r, w2t, b2r))
    parts = []
    eip, ejp = ei, ej
    for p in range(_NP):
        ep = es[p]
        if p:
            pu, ps = parts[-1]
            pu, ps, eip, ejp, ep = lax.optimization_barrier(
                (pu, ps, ei, ej, ep))
            parts[-1] = (pu, ps)
        parts.append(_make_sc_call(p)(eip, ejp, ep, h0, h1, zeros))
    flat = [a for pair in parts for a in pair]
    return _fin_call(*flat)
